# initial kernel scaffold (unmeasured)
import jax
import jax.numpy as jnp
from jax import lax
from jax.experimental import pallas as pl
from jax.experimental.pallas import tpu as pltpu


def kernel(
    x,
):
    def body(*refs):
        pass

    out_shape = jax.ShapeDtypeStruct(..., jnp.float32)
    return pl.pallas_call(body, out_shape=out_shape)(...)


# baseline (device time: 27601 ns/iter reference)
import jax
import jax.numpy as jnp
from jax import lax
from jax.experimental import pallas as pl
from jax.experimental.pallas import tpu as pltpu

N_DEV = 8


def kernel(x):
    m, n = x.shape
    mc = m // N_DEV

    def body(x_ref, out_ref, send_buf, rs_buf, p1_send, p1_recv, p2_send, p2_recv):
        my_i = lax.axis_index("i")

        p1 = []
        for o in range(1, N_DEV):
            t = (my_i + o) % N_DEV
            k = o - 1
            send_buf[k, :, :] = x_ref[pl.ds(t * mc, mc), :].astype(jnp.bfloat16)
            rdma = pltpu.make_async_remote_copy(
                src_ref=send_buf.at[k],
                dst_ref=rs_buf.at[k],
                send_sem=p1_send.at[k],
                recv_sem=p1_recv.at[k],
                device_id=(t,),
                device_id_type=pl.DeviceIdType.MESH,
            )
            rdma.start()
            p1.append(rdma)

        acc = x_ref[pl.ds(my_i * mc, mc), :]
        for k in range(N_DEV - 1):
            p1[k].wait_recv()
            acc = acc + rs_buf[k, :, :].astype(jnp.float32)

        out_ref[pl.ds(my_i * mc, mc), :] = acc.astype(jnp.bfloat16)

        p2 = []
        for o in range(1, N_DEV):
            t = (my_i + o) % N_DEV
            k = o - 1
            rdma = pltpu.make_async_remote_copy(
                src_ref=out_ref.at[pl.ds(my_i * mc, mc), :],
                dst_ref=out_ref.at[pl.ds(my_i * mc, mc), :],
                send_sem=p2_send.at[k],
                recv_sem=p2_recv.at[k],
                device_id=(t,),
                device_id_type=pl.DeviceIdType.MESH,
            )
            rdma.start()
            p2.append(rdma)

        for r in p1:
            r.wait_send()
        for r in p2:
            r.wait_recv()
        for r in p2:
            r.wait_send()

    out_shape = jax.ShapeDtypeStruct((m, n), jnp.bfloat16)
    return pl.pallas_call(
        body,
        out_shape=out_shape,
        in_specs=[pl.BlockSpec(memory_space=pltpu.VMEM)],
        out_specs=pl.BlockSpec(memory_space=pltpu.VMEM),
        scratch_shapes=[
            pltpu.VMEM((N_DEV - 1, mc, n), jnp.bfloat16),
            pltpu.VMEM((N_DEV - 1, mc, n), jnp.bfloat16),
            pltpu.SemaphoreType.DMA((N_DEV - 1,)),
            pltpu.SemaphoreType.DMA((N_DEV - 1,)),
            pltpu.SemaphoreType.DMA((N_DEV - 1,)),
            pltpu.SemaphoreType.DMA((N_DEV - 1,)),
        ],
    )(x)


# device time: 23716 ns/iter; 1.1638x vs baseline; 1.1638x over previous
import jax
import jax.numpy as jnp
from jax import lax
from jax.experimental import pallas as pl
from jax.experimental.pallas import tpu as pltpu

N_DEV = 8
NS = 2


def kernel(x):
    m, n = x.shape
    mc = m // N_DEV
    nw = n // NS

    def body(x_ref, out_ref, x16_ref, rs_buf, p1_send, p1_recv, p2_send, p2_recv):
        my_i = lax.axis_index("i")

        x16_ref[:, :] = x_ref[:, :].astype(jnp.bfloat16)

        p1 = []
        for s in range(NS):
            for o in range(1, N_DEV):
                t = (my_i + o) % N_DEV
                k = o - 1
                rdma = pltpu.make_async_remote_copy(
                    src_ref=x16_ref.at[pl.ds(t * mc, mc), pl.ds(s * nw, nw)],
                    dst_ref=rs_buf.at[s, k],
                    send_sem=p1_send.at[s, k],
                    recv_sem=p1_recv.at[s, k],
                    device_id=(t,),
                    device_id_type=pl.DeviceIdType.MESH,
                )
                rdma.start()
                p1.append(rdma)

        p2 = []
        for s in range(NS):
            acc = x_ref[pl.ds(my_i * mc, mc), pl.ds(s * nw, nw)]
            for k in range(N_DEV - 1):
                p1[s * (N_DEV - 1) + k].wait_recv()
                acc = acc + rs_buf[s, k, :, :].astype(jnp.float32)
            out_ref[pl.ds(my_i * mc, mc), pl.ds(s * nw, nw)] = acc.astype(
                jnp.bfloat16
            )
            for o in range(1, N_DEV):
                t = (my_i + o) % N_DEV
                k = o - 1
                rdma = pltpu.make_async_remote_copy(
                    src_ref=out_ref.at[pl.ds(my_i * mc, mc), pl.ds(s * nw, nw)],
                    dst_ref=out_ref.at[pl.ds(my_i * mc, mc), pl.ds(s * nw, nw)],
                    send_sem=p2_send.at[s, k],
                    recv_sem=p2_recv.at[s, k],
                    device_id=(t,),
                    device_id_type=pl.DeviceIdType.MESH,
                )
                rdma.start()
                p2.append(rdma)

        for r in p1:
            r.wait_send()
        for r in p2:
            r.wait_recv()
        for r in p2:
            r.wait_send()

    out_shape = jax.ShapeDtypeStruct((m, n), jnp.bfloat16)
    return pl.pallas_call(
        body,
        out_shape=out_shape,
        in_specs=[pl.BlockSpec(memory_space=pltpu.VMEM)],
        out_specs=pl.BlockSpec(memory_space=pltpu.VMEM),
        scratch_shapes=[
            pltpu.VMEM((m, n), jnp.bfloat16),
            pltpu.VMEM((NS, N_DEV - 1, mc, nw), jnp.bfloat16),
            pltpu.SemaphoreType.DMA((NS, N_DEV - 1)),
            pltpu.SemaphoreType.DMA((NS, N_DEV - 1)),
            pltpu.SemaphoreType.DMA((NS, N_DEV - 1)),
            pltpu.SemaphoreType.DMA((NS, N_DEV - 1)),
        ],
    )(x)
